# baseline (speedup 1.0000x reference)
import functools

import jax
import jax.numpy as jnp
from jax import lax
from jax.experimental import pallas as pl
from jax.experimental.pallas import tpu as pltpu
from jax.experimental.pallas import tpu_sc as plsc

_N = 10000
_E = 320000
_D_IN = 128
_H = 64
_G = 16
_OUT = 2

_NC = 2
_NS = 16
_NW = _NC * _NS
_CHUNK = 128
_NCHUNK = 80
_NROWS = _E // _CHUNK
_LASTN = _NROWS - (_NW - 1) * _NCHUNK
_NBUF = 2
_NP = 10240
_RPT = _NP // _NS
_DEGW = 8

_PREC = jax.lax.Precision.DEFAULT


def _rsqrt(x):
  r = lax.rsqrt(x)
  return r * (1.5 - 0.5 * x * r * r)


def _load_slab(idx_hbm, idx_v, w):
  @pl.when(w < _NW - 1)
  def _():
    pltpu.sync_copy(idx_hbm.at[pl.ds(w * _NCHUNK, _NCHUNK)], idx_v)

  @pl.when(w == _NW - 1)
  def _():
    pltpu.sync_copy(idx_hbm.at[pl.ds((_NW - 1) * _NCHUNK, _LASTN)],
                    idx_v.at[pl.ds(0, _LASTN)])


def _nchunks(w):
  return lax.select(w == _NW - 1, _LASTN, _NCHUNK)


def _deg_body(dst_hbm, ones_hbm, zeros_hbm, out_hbm, idx_v, ones_v, acc):
  c = lax.axis_index("c")
  s = lax.axis_index("s")
  w = s * _NC + c
  pltpu.sync_copy(ones_hbm, ones_v)
  pltpu.sync_copy(zeros_hbm, acc.at[pl.ds(s * _RPT, _RPT)])
  plsc.subcore_barrier()
  _load_slab(dst_hbm, idx_v, w)

  def step(i, carry):
    pltpu.sync_copy(ones_v, acc.at[idx_v.at[i]], add=True)
    return carry

  lax.fori_loop(0, _nchunks(w), step, 0)
  plsc.subcore_barrier()
  pltpu.sync_copy(acc.at[pl.ds(s * _RPT, _RPT)],
                  out_hbm.at[c, pl.ds(s * _RPT, _RPT)])


def _msg_body(h_hbm, src_hbm, dst_hbm, zeros_hbm, out_hbm,
              sidx_v, didx_v, r0, r1, h_sh, acc, g0, g1, ss0, ss1):
  c = lax.axis_index("c")
  s = lax.axis_index("s")
  w = s * _NC + c
  rows = (r0, r1)
  gsems = (g0, g1)
  ssems = (ss0, ss1)
  pltpu.sync_copy(zeros_hbm, acc.at[pl.ds(s * _RPT, _RPT)])

  @pl.when(s < _NS - 1)
  def _():
    pltpu.sync_copy(h_hbm.at[pl.ds(s * _RPT, _RPT)],
                    h_sh.at[pl.ds(s * _RPT, _RPT)])

  @pl.when(s == _NS - 1)
  def _():
    pltpu.sync_copy(h_hbm.at[pl.ds((_NS - 1) * _RPT, _N - (_NS - 1) * _RPT)],
                    h_sh.at[pl.ds((_NS - 1) * _RPT, _N - (_NS - 1) * _RPT)])

  plsc.subcore_barrier()
  _load_slab(src_hbm, sidx_v, w)
  _load_slab(dst_hbm, didx_v, w)

  nck = _nchunks(w) // _NBUF

  for b in range(_NBUF):
    pltpu.async_copy(h_sh.at[sidx_v.at[b]], rows[b], gsems[b])

  def step(k, carry):
    for b in range(_NBUF):
      i = k * _NBUF + b
      pltpu.make_async_copy(h_sh.at[sidx_v.at[i]], rows[b], gsems[b]).wait()
      pltpu.async_copy(rows[b], acc.at[didx_v.at[i]], ssems[b], add=True)
    for b in range(_NBUF):
      i = k * _NBUF + b
      pltpu.make_async_copy(rows[b], acc.at[didx_v.at[i]], ssems[b]).wait()

      @pl.when(k < nck - 1)
      def _():
        pltpu.async_copy(h_sh.at[sidx_v.at[i + _NBUF]], rows[b], gsems[b])

    return carry

  lax.fori_loop(0, nck, step, 0)
  plsc.subcore_barrier()
  pltpu.sync_copy(acc.at[pl.ds(s * _RPT, _RPT)],
                  out_hbm.at[c, pl.ds(s * _RPT, _RPT)])


@functools.cache
def _sc_mesh():
  return plsc.VectorSubcoreMesh(core_axis_name="c", subcore_axis_name="s",
                                num_cores=_NC, num_subcores=_NS)


@functools.cache
def _deg_call():
  return pl.kernel(
      _deg_body,
      out_type=jax.ShapeDtypeStruct((_NC, _NP, _DEGW), jnp.float32),
      mesh=_sc_mesh(),
      compiler_params=pltpu.CompilerParams(use_tc_tiling_on_sc=False),
      scratch_types=[
          pltpu.VMEM((_NCHUNK, _CHUNK), jnp.int32),
          pltpu.VMEM((_CHUNK, _DEGW), jnp.float32),
          pltpu.VMEM_SHARED((_NP, _DEGW), jnp.float32),
      ],
  )


@functools.cache
def _msg_call():
  return pl.kernel(
      _msg_body,
      out_type=jax.ShapeDtypeStruct((_NC, _NP, _H), jnp.float32),
      mesh=_sc_mesh(),
      compiler_params=pltpu.CompilerParams(use_tc_tiling_on_sc=False),
      scratch_types=[
          pltpu.VMEM((_NCHUNK, _CHUNK), jnp.int32),
          pltpu.VMEM((_NCHUNK, _CHUNK), jnp.int32),
          pltpu.VMEM((_CHUNK, _H), jnp.float32),
          pltpu.VMEM((_CHUNK, _H), jnp.float32),
          pltpu.VMEM_SHARED((_N, _H), jnp.float32),
          pltpu.VMEM_SHARED((_NP, _H), jnp.float32),
          pltpu.SemaphoreType.DMA,
          pltpu.SemaphoreType.DMA,
          pltpu.SemaphoreType.DMA,
          pltpu.SemaphoreType.DMA,
      ],
  )


def _k2a_body(x_ref, w1_ref, h1_ref):
  h1_ref[...] = jnp.dot(x_ref[...], w1_ref[...],
                        preferred_element_type=jnp.float32, precision=_PREC)


def _k2b_body(h1_ref, degp_ref, h1p_ref, inv_ref):
  deg = degp_ref[0, 0:_N, 0:1] + degp_ref[1, 0:_N, 0:1] + 1.0
  inv = _rsqrt(deg)
  h1p_ref[...] = h1_ref[...] * inv
  inv_ref[...] = inv


def _k4_body(p_ref, h1p_ref, inv_ref, b1_ref, g1_ref, be1_ref,
             w2_ref, h2p_ref):
  inv = inv_ref[...]
  m = (p_ref[0, 0:_N, :] + p_ref[1, 0:_N, :] + h1p_ref[...]) * inv + b1_ref[...]
  mean = jnp.mean(m, axis=0, keepdims=True)
  cen = m - mean
  var = jnp.mean(cen * cen, axis=0, keepdims=True)
  bn = cen * _rsqrt(var + 1e-5) * g1_ref[...] + be1_ref[...]
  r = jnp.maximum(bn, 0.0)
  h2 = jnp.dot(r, w2_ref[...], preferred_element_type=jnp.float32,
               precision=_PREC)
  h2p_ref[...] = h2 * inv


def _k6_body(p_ref, h2p_ref, inv_ref, batch_ref, b2_ref, g2_ref,
             be2_ref, wc1_ref, bc1_ref, wc2_ref, bc2_ref, out_ref):
  inv = inv_ref[...]
  m = (p_ref[0, 0:_N, :] + p_ref[1, 0:_N, :] + h2p_ref[...]) * inv + b2_ref[...]
  mean = jnp.mean(m, axis=0, keepdims=True)
  cen = m - mean
  var = jnp.mean(cen * cen, axis=0, keepdims=True)
  bn = cen * _rsqrt(var + 1e-5) * g2_ref[...] + be2_ref[...]
  h = jnp.maximum(bn, 0.0)
  seg = lax.broadcasted_iota(jnp.int32, (1, _G), 1)
  oh = (batch_ref[...] == seg).astype(jnp.float32)
  sums = lax.dot_general(oh, h, (((0,), (0,)), ((), ())),
                         preferred_element_type=jnp.float32,
                         precision=_PREC)
  counts = jnp.sum(oh, axis=0, keepdims=True)
  pooled = sums * (1.0 / jnp.maximum(jnp.transpose(counts), 1.0))
  z = jnp.maximum(
      jnp.dot(pooled, wc1_ref[...], preferred_element_type=jnp.float32,
              precision=_PREC) + bc1_ref[...], 0.0)
  out_ref[...] = jnp.dot(z, wc2_ref[...], preferred_element_type=jnp.float32,
                         precision=_PREC) + bc2_ref[...]


def _tc_call(body, out_shape):
  return pl.pallas_call(body, out_shape=out_shape)


@jax.jit
def kernel(x, edge_index, batch, W1, b1, g1, be1, W2, b2, g2, be2,
           Wc1, bc1, Wc2, bc2):
  src = edge_index[0].reshape(_NROWS, _CHUNK)
  dst = edge_index[1].reshape(_NROWS, _CHUNK)
  ones_blk = jnp.ones((_CHUNK, _DEGW), jnp.float32)
  zeros_deg = jnp.zeros((_RPT, _DEGW), jnp.float32)
  zeros_msg = jnp.zeros((_RPT, _H), jnp.float32)

  deg_p = _deg_call()(dst, ones_blk, zeros_deg)

  h1 = _tc_call(
      _k2a_body, jax.ShapeDtypeStruct((_N, _H), jnp.float32))(x, W1)

  h1p, inv = _tc_call(
      _k2b_body,
      [jax.ShapeDtypeStruct((_N, _H), jnp.float32),
       jax.ShapeDtypeStruct((_N, 1), jnp.float32)],
  )(h1, deg_p)

  p1 = _msg_call()(h1p, src, dst, zeros_msg)

  h2p = _tc_call(
      _k4_body, jax.ShapeDtypeStruct((_N, _H), jnp.float32),
  )(p1, h1p, inv, b1.reshape(1, _H), g1.reshape(1, _H),
    be1.reshape(1, _H), W2)

  p2 = _msg_call()(h2p, src, dst, zeros_msg)

  logits = _tc_call(
      _k6_body, jax.ShapeDtypeStruct((_G, _OUT), jnp.float32),
  )(p2, h2p, inv, batch.reshape(_N, 1), b2.reshape(1, _H),
    g2.reshape(1, _H), be2.reshape(1, _H), Wc1, bc1.reshape(1, _H // 2),
    Wc2, bc2.reshape(1, _OUT))
  return logits


# speedup vs baseline: 1.0023x; 1.0023x over previous
import functools

import jax
import jax.numpy as jnp
from jax import lax
from jax.experimental import pallas as pl
from jax.experimental.pallas import tpu as pltpu
from jax.experimental.pallas import tpu_sc as plsc

_N = 10000
_E = 320000
_D_IN = 128
_H = 64
_G = 16
_OUT = 2

_NC = 2
_NS = 16
_NW = _NC * _NS
_CHUNK = 128
_NCHUNK = 80
_NROWS = _E // _CHUNK
_LASTN = _NROWS - (_NW - 1) * _NCHUNK
_NBUF = 2
_NP = 10240
_RPT = _NP // _NS
_DEGW = 8

_PREC = jax.lax.Precision.DEFAULT


def _rsqrt(x):
  r = lax.rsqrt(x)
  return r * (1.5 - 0.5 * x * r * r)


def _load_slab(idx_hbm, idx_v, w):
  @pl.when(w < _NW - 1)
  def _():
    pltpu.sync_copy(idx_hbm.at[pl.ds(w * _NCHUNK, _NCHUNK)], idx_v)

  @pl.when(w == _NW - 1)
  def _():
    pltpu.sync_copy(idx_hbm.at[pl.ds((_NW - 1) * _NCHUNK, _LASTN)],
                    idx_v.at[pl.ds(0, _LASTN)])


def _nchunks(w):
  return lax.select(w == _NW - 1, _LASTN, _NCHUNK)


def _deg_body(dst_hbm, ones_hbm, zeros_hbm, out_hbm, idx_v, ones_v, acc):
  c = lax.axis_index("c")
  s = lax.axis_index("s")
  w = s * _NC + c
  pltpu.sync_copy(ones_hbm, ones_v)
  pltpu.sync_copy(zeros_hbm, acc.at[pl.ds(s * _RPT, _RPT)])
  plsc.subcore_barrier()
  _load_slab(dst_hbm, idx_v, w)

  def step(i, carry):
    pltpu.sync_copy(ones_v, acc.at[idx_v.at[i]], add=True)
    return carry

  lax.fori_loop(0, _nchunks(w), step, 0)
  plsc.subcore_barrier()
  pltpu.sync_copy(acc.at[pl.ds(s * _RPT, _RPT)],
                  out_hbm.at[c, pl.ds(s * _RPT, _RPT)])


def _msg_body(h_hbm, src_hbm, dst_hbm, zeros_hbm, out_hbm,
              sidx_v, didx_v, r0, r1, h_sh, acc, g0, g1, ss0, ss1):
  c = lax.axis_index("c")
  s = lax.axis_index("s")
  w = s * _NC + c
  rows = (r0, r1)
  gsems = (g0, g1)
  ssems = (ss0, ss1)
  pltpu.sync_copy(zeros_hbm, acc.at[pl.ds(s * _RPT, _RPT)])

  @pl.when(s < _NS - 1)
  def _():
    pltpu.sync_copy(h_hbm.at[pl.ds(s * _RPT, _RPT)],
                    h_sh.at[pl.ds(s * _RPT, _RPT)])

  @pl.when(s == _NS - 1)
  def _():
    pltpu.sync_copy(h_hbm.at[pl.ds((_NS - 1) * _RPT, _N - (_NS - 1) * _RPT)],
                    h_sh.at[pl.ds((_NS - 1) * _RPT, _N - (_NS - 1) * _RPT)])

  plsc.subcore_barrier()
  _load_slab(src_hbm, sidx_v, w)
  _load_slab(dst_hbm, didx_v, w)

  nck = _nchunks(w) // _NBUF

  for b in range(_NBUF):
    pltpu.async_copy(h_sh.at[sidx_v.at[b]], rows[b], gsems[b])

  def step(k, carry):
    for b in range(_NBUF):
      i = k * _NBUF + b
      pltpu.make_async_copy(h_sh.at[sidx_v.at[i]], rows[b], gsems[b]).wait()
      pltpu.async_copy(rows[b], acc.at[didx_v.at[i]], ssems[b], add=True)
    for b in range(_NBUF):
      i = k * _NBUF + b
      pltpu.make_async_copy(rows[b], acc.at[didx_v.at[i]], ssems[b]).wait()

      @pl.when(k < nck - 1)
      def _():
        pltpu.async_copy(h_sh.at[sidx_v.at[i + _NBUF]], rows[b], gsems[b])

    return carry

  lax.fori_loop(0, nck, step, 0)
  plsc.subcore_barrier()
  pltpu.sync_copy(acc.at[pl.ds(s * _RPT, _RPT)],
                  out_hbm.at[c, pl.ds(s * _RPT, _RPT)])


@functools.cache
def _sc_mesh():
  return plsc.VectorSubcoreMesh(core_axis_name="c", subcore_axis_name="s",
                                num_cores=_NC, num_subcores=_NS)


@functools.cache
def _deg_call():
  return pl.kernel(
      _deg_body,
      out_type=jax.ShapeDtypeStruct((_NC, _NP, _DEGW), jnp.float32),
      mesh=_sc_mesh(),
      compiler_params=pltpu.CompilerParams(use_tc_tiling_on_sc=False),
      scratch_types=[
          pltpu.VMEM((_NCHUNK, _CHUNK), jnp.int32),
          pltpu.VMEM((_CHUNK, _DEGW), jnp.float32),
          pltpu.VMEM_SHARED((_NP, _DEGW), jnp.float32),
      ],
  )


@functools.cache
def _msg_call():
  return pl.kernel(
      _msg_body,
      out_type=jax.ShapeDtypeStruct((_NC, _NP, _H), jnp.float32),
      mesh=_sc_mesh(),
      compiler_params=pltpu.CompilerParams(use_tc_tiling_on_sc=False),
      scratch_types=[
          pltpu.VMEM((_NCHUNK, _CHUNK), jnp.int32),
          pltpu.VMEM((_NCHUNK, _CHUNK), jnp.int32),
          pltpu.VMEM((_CHUNK, _H), jnp.float32),
          pltpu.VMEM((_CHUNK, _H), jnp.float32),
          pltpu.VMEM_SHARED((_N, _H), jnp.float32),
          pltpu.VMEM_SHARED((_NP, _H), jnp.float32),
          pltpu.SemaphoreType.DMA,
          pltpu.SemaphoreType.DMA,
          pltpu.SemaphoreType.DMA,
          pltpu.SemaphoreType.DMA,
      ],
  )


def _k2_body(x_ref, w1_ref, degp_ref, h1p_ref, inv_ref):
  deg = degp_ref[0, 0:_N, 0:1] + degp_ref[1, 0:_N, 0:1] + 1.0
  inv = _rsqrt(deg)
  h = jnp.dot(x_ref[...], w1_ref[...], preferred_element_type=jnp.float32,
              precision=_PREC)
  h1p_ref[...] = h * inv
  inv_ref[...] = inv


def _k4_body(p_ref, h1p_ref, inv_ref, b1_ref, g1_ref, be1_ref,
             w2_ref, h2p_ref):
  inv = inv_ref[...]
  m = (p_ref[0, 0:_N, :] + p_ref[1, 0:_N, :] + h1p_ref[...]) * inv + b1_ref[...]
  mean = jnp.mean(m, axis=0, keepdims=True)
  cen = m - mean
  var = jnp.mean(cen * cen, axis=0, keepdims=True)
  bn = cen * _rsqrt(var + 1e-5) * g1_ref[...] + be1_ref[...]
  r = jnp.maximum(bn, 0.0)
  h2 = jnp.dot(r, w2_ref[...], preferred_element_type=jnp.float32,
               precision=_PREC)
  h2p_ref[...] = h2 * inv


def _k6_body(p_ref, h2p_ref, inv_ref, batch_ref, b2_ref, g2_ref,
             be2_ref, wc1_ref, bc1_ref, wc2_ref, bc2_ref, out_ref):
  inv = inv_ref[...]
  m = (p_ref[0, 0:_N, :] + p_ref[1, 0:_N, :] + h2p_ref[...]) * inv + b2_ref[...]
  mean = jnp.mean(m, axis=0, keepdims=True)
  cen = m - mean
  var = jnp.mean(cen * cen, axis=0, keepdims=True)
  bn = cen * _rsqrt(var + 1e-5) * g2_ref[...] + be2_ref[...]
  h = jnp.maximum(bn, 0.0)
  seg = lax.broadcasted_iota(jnp.int32, (1, _G), 1)
  oh = (batch_ref[...] == seg).astype(jnp.float32)
  sums = lax.dot_general(oh, h, (((0,), (0,)), ((), ())),
                         preferred_element_type=jnp.float32,
                         precision=_PREC)
  counts = jnp.sum(oh, axis=0, keepdims=True)
  pooled = sums * (1.0 / jnp.maximum(jnp.transpose(counts), 1.0))
  z = jnp.maximum(
      jnp.dot(pooled, wc1_ref[...], preferred_element_type=jnp.float32,
              precision=_PREC) + bc1_ref[...], 0.0)
  out_ref[...] = jnp.dot(z, wc2_ref[...], preferred_element_type=jnp.float32,
                         precision=_PREC) + bc2_ref[...]


def _tc_call(body, out_shape):
  return pl.pallas_call(body, out_shape=out_shape)


@jax.jit
def kernel(x, edge_index, batch, W1, b1, g1, be1, W2, b2, g2, be2,
           Wc1, bc1, Wc2, bc2):
  src = edge_index[0].reshape(_NROWS, _CHUNK)
  dst = edge_index[1].reshape(_NROWS, _CHUNK)
  ones_blk = jnp.ones((_CHUNK, _DEGW), jnp.float32)
  zeros_deg = jnp.zeros((_RPT, _DEGW), jnp.float32)
  zeros_msg = jnp.zeros((_RPT, _H), jnp.float32)

  deg_p = _deg_call()(dst, ones_blk, zeros_deg)

  h1p, inv = _tc_call(
      _k2_body,
      [jax.ShapeDtypeStruct((_N, _H), jnp.float32),
       jax.ShapeDtypeStruct((_N, 1), jnp.float32)],
  )(x, W1, deg_p)

  p1 = _msg_call()(h1p, src, dst, zeros_msg)

  h2p = _tc_call(
      _k4_body, jax.ShapeDtypeStruct((_N, _H), jnp.float32),
  )(p1, h1p, inv, b1.reshape(1, _H), g1.reshape(1, _H),
    be1.reshape(1, _H), W2)

  p2 = _msg_call()(h2p, src, dst, zeros_msg)

  logits = _tc_call(
      _k6_body, jax.ShapeDtypeStruct((_G, _OUT), jnp.float32),
  )(p2, h2p, inv, batch.reshape(_N, 1), b2.reshape(1, _H),
    g2.reshape(1, _H), be2.reshape(1, _H), Wc1, bc1.reshape(1, _H // 2),
    Wc2, bc2.reshape(1, _OUT))
  return logits


# speedup vs baseline: 1.0224x; 1.0200x over previous
import functools

import jax
import jax.numpy as jnp
from jax import lax
from jax.experimental import pallas as pl
from jax.experimental.pallas import tpu as pltpu
from jax.experimental.pallas import tpu_sc as plsc

_N = 10000
_E = 320000
_D_IN = 128
_H = 64
_G = 16
_OUT = 2

_NC = 2
_NS = 16
_NW = _NC * _NS
_CHUNK = 128
_NCHUNK = 80
_NROWS = _E // _CHUNK
_LASTN = _NROWS - (_NW - 1) * _NCHUNK
_NBUF = 3
_NP = 10240
_RPT = _NP // _NS
_DEGW = 8

_PREC = jax.lax.Precision.DEFAULT


def _rsqrt(x):
  r = lax.rsqrt(x)
  return r * (1.5 - 0.5 * x * r * r)


def _load_slab(idx_hbm, idx_v, w):
  @pl.when(w < _NW - 1)
  def _():
    pltpu.sync_copy(idx_hbm.at[pl.ds(w * _NCHUNK, _NCHUNK)], idx_v)

  @pl.when(w == _NW - 1)
  def _():
    pltpu.sync_copy(idx_hbm.at[pl.ds((_NW - 1) * _NCHUNK, _LASTN)],
                    idx_v.at[pl.ds(0, _LASTN)])


def _nchunks(w):
  return lax.select(w == _NW - 1, _LASTN, _NCHUNK)


def _deg_body(dst_hbm, ones_hbm, zeros_hbm, out_hbm, idx_v, ones_v, acc):
  c = lax.axis_index("c")
  s = lax.axis_index("s")
  w = s * _NC + c
  pltpu.sync_copy(ones_hbm, ones_v)
  pltpu.sync_copy(zeros_hbm, acc.at[pl.ds(s * _RPT, _RPT)])
  plsc.subcore_barrier()
  _load_slab(dst_hbm, idx_v, w)

  def step(i, carry):
    pltpu.sync_copy(ones_v, acc.at[idx_v.at[i]], add=True)
    return carry

  lax.fori_loop(0, _nchunks(w), step, 0)
  plsc.subcore_barrier()
  pltpu.sync_copy(acc.at[pl.ds(s * _RPT, _RPT)],
                  out_hbm.at[c, pl.ds(s * _RPT, _RPT)])


def _msg_body(h_hbm, src_hbm, dst_hbm, zeros_hbm, out_hbm,
              sidx_v, didx_v, r0, r1, r2, h_sh, acc, g0, g1, g2):
  c = lax.axis_index("c")
  s = lax.axis_index("s")
  w = s * _NC + c
  rows = (r0, r1, r2)
  gsems = (g0, g1, g2)
  pltpu.sync_copy(zeros_hbm, acc.at[pl.ds(s * _RPT, _RPT)])

  @pl.when(s < _NS - 1)
  def _():
    pltpu.sync_copy(h_hbm.at[pl.ds(s * _RPT, _RPT)],
                    h_sh.at[pl.ds(s * _RPT, _RPT)])

  @pl.when(s == _NS - 1)
  def _():
    pltpu.sync_copy(h_hbm.at[pl.ds((_NS - 1) * _RPT, _N - (_NS - 1) * _RPT)],
                    h_sh.at[pl.ds((_NS - 1) * _RPT, _N - (_NS - 1) * _RPT)])

  plsc.subcore_barrier()
  _load_slab(src_hbm, sidx_v, w)
  _load_slab(dst_hbm, didx_v, w)

  nch = _nchunks(w)

  for b in range(_NBUF):
    pltpu.async_copy(h_sh.at[sidx_v.at[b]], rows[b], gsems[b])

  def step(k, carry):
    for b in range(_NBUF):
      i = k * _NBUF + b

      @pl.when(i < nch)
      def _():
        pltpu.make_async_copy(h_sh.at[sidx_v.at[i]], rows[b], gsems[b]).wait()
        pltpu.sync_copy(rows[b], acc.at[didx_v.at[i]], add=True)

        @pl.when(i + _NBUF < nch)
        def _():
          pltpu.async_copy(h_sh.at[sidx_v.at[i + _NBUF]], rows[b], gsems[b])

    return carry

  lax.fori_loop(0, (nch + _NBUF - 1) // _NBUF, step, 0)
  plsc.subcore_barrier()
  pltpu.sync_copy(acc.at[pl.ds(s * _RPT, _RPT)],
                  out_hbm.at[c, pl.ds(s * _RPT, _RPT)])


@functools.cache
def _sc_mesh():
  return plsc.VectorSubcoreMesh(core_axis_name="c", subcore_axis_name="s",
                                num_cores=_NC, num_subcores=_NS)


@functools.cache
def _deg_call():
  return pl.kernel(
      _deg_body,
      out_type=jax.ShapeDtypeStruct((_NC, _NP, _DEGW), jnp.float32),
      mesh=_sc_mesh(),
      compiler_params=pltpu.CompilerParams(use_tc_tiling_on_sc=False),
      scratch_types=[
          pltpu.VMEM((_NCHUNK, _CHUNK), jnp.int32),
          pltpu.VMEM((_CHUNK, _DEGW), jnp.float32),
          pltpu.VMEM_SHARED((_NP, _DEGW), jnp.float32),
      ],
  )


@functools.cache
def _msg_call():
  return pl.kernel(
      _msg_body,
      out_type=jax.ShapeDtypeStruct((_NC, _NP, _H), jnp.float32),
      mesh=_sc_mesh(),
      compiler_params=pltpu.CompilerParams(use_tc_tiling_on_sc=False),
      scratch_types=[
          pltpu.VMEM((_NCHUNK, _CHUNK), jnp.int32),
          pltpu.VMEM((_NCHUNK, _CHUNK), jnp.int32),
          pltpu.VMEM((_CHUNK, _H), jnp.float32),
          pltpu.VMEM((_CHUNK, _H), jnp.float32),
          pltpu.VMEM((_CHUNK, _H), jnp.float32),
          pltpu.VMEM_SHARED((_N, _H), jnp.float32),
          pltpu.VMEM_SHARED((_NP, _H), jnp.float32),
          pltpu.SemaphoreType.DMA,
          pltpu.SemaphoreType.DMA,
          pltpu.SemaphoreType.DMA,
      ],
  )


def _k2_body(x_ref, w1_ref, degp_ref, h1p_ref, inv_ref):
  deg = degp_ref[0, 0:_N, 0:1] + degp_ref[1, 0:_N, 0:1] + 1.0
  inv = _rsqrt(deg)
  h = jnp.dot(x_ref[...], w1_ref[...], preferred_element_type=jnp.float32,
              precision=_PREC)
  h1p_ref[...] = h * inv
  inv_ref[...] = inv


def _k4_body(p_ref, h1p_ref, inv_ref, b1_ref, g1_ref, be1_ref,
             w2_ref, h2p_ref):
  inv = inv_ref[...]
  m = (p_ref[0, 0:_N, :] + p_ref[1, 0:_N, :] + h1p_ref[...]) * inv + b1_ref[...]
  mean = jnp.mean(m, axis=0, keepdims=True)
  cen = m - mean
  var = jnp.mean(cen * cen, axis=0, keepdims=True)
  bn = cen * _rsqrt(var + 1e-5) * g1_ref[...] + be1_ref[...]
  r = jnp.maximum(bn, 0.0)
  h2 = jnp.dot(r, w2_ref[...], preferred_element_type=jnp.float32,
               precision=_PREC)
  h2p_ref[...] = h2 * inv


def _k6_body(p_ref, h2p_ref, inv_ref, batch_ref, b2_ref, g2_ref,
             be2_ref, wc1_ref, bc1_ref, wc2_ref, bc2_ref, out_ref):
  inv = inv_ref[...]
  m = (p_ref[0, 0:_N, :] + p_ref[1, 0:_N, :] + h2p_ref[...]) * inv + b2_ref[...]
  mean = jnp.mean(m, axis=0, keepdims=True)
  cen = m - mean
  var = jnp.mean(cen * cen, axis=0, keepdims=True)
  bn = cen * _rsqrt(var + 1e-5) * g2_ref[...] + be2_ref[...]
  h = jnp.maximum(bn, 0.0)
  seg = lax.broadcasted_iota(jnp.int32, (1, _G), 1)
  oh = (batch_ref[...] == seg).astype(jnp.float32)
  sums = lax.dot_general(oh, h, (((0,), (0,)), ((), ())),
                         preferred_element_type=jnp.float32,
                         precision=_PREC)
  counts = jnp.sum(oh, axis=0, keepdims=True)
  pooled = sums * (1.0 / jnp.maximum(jnp.transpose(counts), 1.0))
  z = jnp.maximum(
      jnp.dot(pooled, wc1_ref[...], preferred_element_type=jnp.float32,
              precision=_PREC) + bc1_ref[...], 0.0)
  out_ref[...] = jnp.dot(z, wc2_ref[...], preferred_element_type=jnp.float32,
                         precision=_PREC) + bc2_ref[...]


def _tc_call(body, out_shape):
  return pl.pallas_call(body, out_shape=out_shape)


@jax.jit
def kernel(x, edge_index, batch, W1, b1, g1, be1, W2, b2, g2, be2,
           Wc1, bc1, Wc2, bc2):
  src = edge_index[0].reshape(_NROWS, _CHUNK)
  dst = edge_index[1].reshape(_NROWS, _CHUNK)
  ones_blk = jnp.ones((_CHUNK, _DEGW), jnp.float32)
  zeros_deg = jnp.zeros((_RPT, _DEGW), jnp.float32)
  zeros_msg = jnp.zeros((_RPT, _H), jnp.float32)

  deg_p = _deg_call()(dst, ones_blk, zeros_deg)

  h1p, inv = _tc_call(
      _k2_body,
      [jax.ShapeDtypeStruct((_N, _H), jnp.float32),
       jax.ShapeDtypeStruct((_N, 1), jnp.float32)],
  )(x, W1, deg_p)

  p1 = _msg_call()(h1p, src, dst, zeros_msg)

  h2p = _tc_call(
      _k4_body, jax.ShapeDtypeStruct((_N, _H), jnp.float32),
  )(p1, h1p, inv, b1.reshape(1, _H), g1.reshape(1, _H),
    be1.reshape(1, _H), W2)

  p2 = _msg_call()(h2p, src, dst, zeros_msg)

  logits = _tc_call(
      _k6_body, jax.ShapeDtypeStruct((_G, _OUT), jnp.float32),
  )(p2, h2p, inv, batch.reshape(_N, 1), b2.reshape(1, _H),
    g2.reshape(1, _H), be2.reshape(1, _H), Wc1, bc1.reshape(1, _H // 2),
    Wc2, bc2.reshape(1, _OUT))
  return logits
